# baseline (speedup 1.0000x reference)
import dataclasses
import functools

import jax
import jax.numpy as jnp
from jax import lax
from jax.experimental import pallas as pl
from jax.experimental.pallas import tpu as pltpu
from jax.experimental.pallas import tpu_sc as plsc

DIM = 32
LANES = 16
NUM_CORES = 2
NUM_SUBCORES = 16
NW = NUM_CORES * NUM_SUBCORES


def kernel(users, items, user_emb, item_emb):
    batch = users.shape[0]
    b_per_w = batch // NW
    mesh = plsc.VectorSubcoreMesh(core_axis_name="c", subcore_axis_name="s")
    cp = pltpu.CompilerParams(
        needs_layout_passes=False, use_tc_tiling_on_sc=False
    )

    @functools.partial(
        pl.kernel,
        compiler_params=cp,
        out_type=jax.ShapeDtypeStruct((batch,), jnp.float32),
        mesh=mesh,
        scratch_types=[
            pltpu.VMEM((b_per_w,), jnp.int32),
            pltpu.VMEM((b_per_w,), jnp.int32),
            pltpu.VMEM((b_per_w, DIM), jnp.float32),
            pltpu.VMEM((b_per_w, DIM), jnp.float32),
            pltpu.VMEM((b_per_w,), jnp.float32),
            pltpu.SemaphoreType.DMA,
            pltpu.SemaphoreType.DMA,
        ],
    )
    def sc_kernel(users_hbm, items_hbm, uemb_hbm, vemb_hbm, out_hbm,
                  uidx_v, iidx_v, urows_v, vrows_v, out_v, sem_u, sem_v):
        wid = lax.axis_index("s") * NUM_CORES + lax.axis_index("c")
        base = wid * b_per_w
        pltpu.sync_copy(users_hbm.at[pl.ds(base, b_per_w)], uidx_v)
        pltpu.sync_copy(items_hbm.at[pl.ds(base, b_per_w)], iidx_v)
        cu = pltpu.async_copy(uemb_hbm.at[uidx_v], urows_v, sem_u)
        cv = pltpu.async_copy(vemb_hbm.at[iidx_v], vrows_v, sem_v)
        cu.wait()
        cv.wait()

        lane = lax.iota(jnp.int32, LANES)

        @pl.loop(0, b_per_w // LANES)
        def _(g):
            r0 = g * LANES
            rows = r0 + lane
            acc = None
            for k in range(DIM):
                col = jnp.full((LANES,), k, jnp.int32)
                u = plsc.load_gather(urows_v, [rows, col])
                v = plsc.load_gather(vrows_v, [rows, col])
                acc = u * v if acc is None else acc + u * v
            out_v[pl.ds(r0, LANES)] = acc

        pltpu.sync_copy(out_v, out_hbm.at[pl.ds(base, b_per_w)])

    return sc_kernel(users, items, user_emb, item_emb)
